# baseline (speedup 1.0000x reference)
import functools

import jax
import jax.numpy as jnp
from jax import lax
from jax.experimental import pallas as pl
from jax.experimental.pallas import tpu as pltpu
from jax.experimental.pallas import tpu_sc as plsc

_NUM_CORES = 2
_NUM_SUBCORES = 16
_LANES = 16


def _embed_kernel(B, S, E, CP):
    NW = _NUM_CORES * _NUM_SUBCORES
    SP = S // NW

    mesh = plsc.VectorSubcoreMesh(core_axis_name="c", subcore_axis_name="s")

    @functools.partial(
        pl.kernel,
        mesh=mesh,
        out_type=jax.ShapeDtypeStruct((B, S, E), jnp.float32),
        scratch_types=[
            pltpu.VMEM((B, SP), jnp.int32),
            pltpu.VMEM((B * CP, E), jnp.float32),
            pltpu.VMEM((CP, E), jnp.float32),
            pltpu.SemaphoreType.DMA,
        ],
    )
    def k(x_hbm, tok_hbm, pos_hbm, out_hbm, idx_v, rows_v, pos_v, sem):
        wid = lax.axis_index("s") * _NUM_CORES + lax.axis_index("c")
        p0 = wid * SP

        for b in range(B):
            pltpu.sync_copy(x_hbm.at[b, pl.ds(p0, SP)], idx_v.at[b])

        @pl.loop(0, SP, step=CP)
        def _chunk(c):
            for b in range(B):
                pltpu.async_copy(
                    tok_hbm.at[idx_v.at[b, pl.ds(c, CP)]],
                    rows_v.at[pl.ds(b * CP, CP)],
                    sem,
                )
            pltpu.sync_copy(pos_hbm.at[pl.ds(p0 + c, CP)], pos_v)
            for b in range(B):
                pltpu.make_async_copy(
                    tok_hbm.at[idx_v.at[b, pl.ds(c, CP)]],
                    rows_v.at[pl.ds(b * CP, CP)],
                    sem,
                ).wait()

            @pl.loop(0, CP)
            def _pos(p):
                @pl.loop(0, E, step=_LANES)
                def _col(e):
                    pv = pos_v.at[p, pl.ds(e, _LANES)][...]
                    for b in range(B):
                        r = rows_v.at[b * CP + p, pl.ds(e, _LANES)]
                        r[...] = r[...] + pv

            for b in range(B):
                pltpu.sync_copy(
                    rows_v.at[pl.ds(b * CP, CP)],
                    out_hbm.at[b, pl.ds(p0 + c, CP)],
                )

    return k


def kernel(x, tok_table, pos_table):
    B, S = x.shape
    _, E = tok_table.shape
    return _embed_kernel(B, S, E, CP=16)(
        x.astype(jnp.int32), tok_table, pos_table
    )


# speedup vs baseline: 1.6695x; 1.6695x over previous
import functools

import jax
import jax.numpy as jnp
from jax import lax
from jax.experimental import pallas as pl
from jax.experimental.pallas import tpu as pltpu
from jax.experimental.pallas import tpu_sc as plsc

_NUM_CORES = 2
_NUM_SUBCORES = 16
_LANES = 16
_NBUF = 3


def _embed_kernel(B, S, E, CP):
    NW = _NUM_CORES * _NUM_SUBCORES
    SP = S // NW
    NCH = SP // CP

    mesh = plsc.VectorSubcoreMesh(core_axis_name="c", subcore_axis_name="s")

    scratch = [pltpu.VMEM((B, SP), jnp.int32)]
    scratch += [pltpu.VMEM((B, CP, E), jnp.float32) for _ in range(_NBUF)]
    scratch += [pltpu.VMEM((CP, E), jnp.float32) for _ in range(_NBUF)]
    scratch += [pltpu.SemaphoreType.DMA for _ in range(3 * _NBUF)]

    @functools.partial(
        pl.kernel,
        mesh=mesh,
        out_type=jax.ShapeDtypeStruct((B, S, E), jnp.float32),
        scratch_types=scratch,
    )
    def k(x_hbm, tok_hbm, pos_hbm, out_hbm, idx_v, *bufs):
        rows = bufs[0:_NBUF]
        pos = bufs[_NBUF:2 * _NBUF]
        sg = bufs[2 * _NBUF:3 * _NBUF]
        sp = bufs[3 * _NBUF:4 * _NBUF]
        so = bufs[4 * _NBUF:5 * _NBUF]

        wid = lax.axis_index("s") * _NUM_CORES + lax.axis_index("c")
        p0 = wid * SP

        for b in range(B):
            pltpu.sync_copy(x_hbm.at[b, pl.ds(p0, SP)], idx_v.at[b])

        def start(c):
            g = c % _NBUF
            for b in range(B):
                pltpu.async_copy(
                    tok_hbm.at[idx_v.at[b, pl.ds(c * CP, CP)]],
                    rows[g].at[b],
                    sg[g],
                )
            pltpu.async_copy(
                pos_hbm.at[pl.ds(p0 + c * CP, CP)], pos[g], sp[g]
            )

        def wait_in(c):
            g = c % _NBUF
            for b in range(B):
                pltpu.make_async_copy(
                    tok_hbm.at[idx_v.at[b, pl.ds(c * CP, CP)]],
                    rows[g].at[b],
                    sg[g],
                ).wait()
            pltpu.make_async_copy(
                pos_hbm.at[pl.ds(p0 + c * CP, CP)], pos[g], sp[g]
            ).wait()

        def wait_out(c):
            g = c % _NBUF
            pltpu.make_async_copy(
                rows[g], out_hbm.at[:, pl.ds(p0 + c * CP, CP)], so[g]
            ).wait()

        def add_and_store(c):
            g = c % _NBUF

            @pl.loop(0, CP)
            def _pos(p):
                @pl.loop(0, E, step=_LANES)
                def _col(e):
                    pv = pos[g].at[p, pl.ds(e, _LANES)][...]
                    for b in range(B):
                        r = rows[g].at[b, p, pl.ds(e, _LANES)]
                        r[...] = r[...] + pv

            pltpu.async_copy(
                rows[g], out_hbm.at[:, pl.ds(p0 + c * CP, CP)], so[g]
            )

        start(0)
        start(1)
        for c in range(NCH):
            wait_in(c)
            add_and_store(c)
            if c + 2 < NCH:
                if c >= 1:
                    wait_out(c - 1)
                start(c + 2)
        wait_out(NCH - 2)
        wait_out(NCH - 1)

    return k


def kernel(x, tok_table, pos_table):
    B, S = x.shape
    _, E = tok_table.shape
    return _embed_kernel(B, S, E, CP=8)(
        x.astype(jnp.int32), tok_table, pos_table
    )


# speedup vs baseline: 2.2619x; 1.3549x over previous
import functools

import jax
import jax.numpy as jnp
from jax import lax
from jax.experimental import pallas as pl
from jax.experimental.pallas import tpu as pltpu
from jax.experimental.pallas import tpu_sc as plsc

_NUM_CORES = 2
_NUM_SUBCORES = 16
_LANES = 16
_NBUF = 3


def _embed_kernel(B, S, E, CP):
    NW = _NUM_CORES * _NUM_SUBCORES
    SP = S // NW
    NCH = SP // CP

    mesh = plsc.VectorSubcoreMesh(core_axis_name="c", subcore_axis_name="s")

    scratch = [pltpu.VMEM((B, SP), jnp.int32)]
    scratch += [pltpu.VMEM((B, CP, E), jnp.float32) for _ in range(_NBUF)]
    scratch += [pltpu.VMEM((CP, E), jnp.float32) for _ in range(_NBUF)]
    scratch += [pltpu.SemaphoreType.DMA for _ in range(3 * _NBUF)]

    @functools.partial(
        pl.kernel,
        mesh=mesh,
        out_type=jax.ShapeDtypeStruct((B, S, E), jnp.float32),
        scratch_types=scratch,
    )
    def k(x_hbm, tok_hbm, pos_hbm, out_hbm, idx_v, *bufs):
        rows = bufs[0:_NBUF]
        pos = bufs[_NBUF:2 * _NBUF]
        sg = bufs[2 * _NBUF:3 * _NBUF]
        sp = bufs[3 * _NBUF:4 * _NBUF]
        so = bufs[4 * _NBUF:5 * _NBUF]

        wid = lax.axis_index("s") * _NUM_CORES + lax.axis_index("c")
        p0 = wid * SP

        for b in range(B):
            pltpu.sync_copy(x_hbm.at[b, pl.ds(p0, SP)], idx_v.at[b])

        def start(c):
            g = c % _NBUF
            for b in range(B):
                pltpu.async_copy(
                    tok_hbm.at[idx_v.at[b, pl.ds(c * CP, CP)]],
                    rows[g].at[b],
                    sg[g],
                )
            pltpu.async_copy(
                pos_hbm.at[pl.ds(p0 + c * CP, CP)], pos[g], sp[g]
            )

        def wait_in(c):
            g = c % _NBUF
            for b in range(B):
                pltpu.make_async_copy(
                    tok_hbm.at[idx_v.at[b, pl.ds(c * CP, CP)]],
                    rows[g].at[b],
                    sg[g],
                ).wait()
            pltpu.make_async_copy(
                pos_hbm.at[pl.ds(p0 + c * CP, CP)], pos[g], sp[g]
            ).wait()

        def wait_out(c):
            g = c % _NBUF
            pltpu.make_async_copy(
                rows[g], out_hbm.at[:, pl.ds(p0 + c * CP, CP)], so[g]
            ).wait()

        def add_and_store(c):
            g = c % _NBUF

            @pl.loop(0, CP)
            def _pos(p):
                @plsc.parallel_loop(0, E, step=_LANES, unroll=4)
                def _col(e):
                    pv = pos[g].at[p, pl.ds(e, _LANES)][...]
                    for b in range(B):
                        r = rows[g].at[b, p, pl.ds(e, _LANES)]
                        r[...] = r[...] + pv

            pltpu.async_copy(
                rows[g], out_hbm.at[:, pl.ds(p0 + c * CP, CP)], so[g]
            )

        start(0)
        start(1)
        for c in range(NCH):
            wait_in(c)
            add_and_store(c)
            if c + 2 < NCH:
                if c >= 1:
                    wait_out(c - 1)
                start(c + 2)
        wait_out(NCH - 2)
        wait_out(NCH - 1)

    return k


def kernel(x, tok_table, pos_table):
    B, S = x.shape
    _, E = tok_table.shape
    return _embed_kernel(B, S, E, CP=8)(
        x.astype(jnp.int32), tok_table, pos_table
    )


# speedup vs baseline: 2.2726x; 1.0047x over previous
import functools

import jax
import jax.numpy as jnp
from jax import lax
from jax.experimental import pallas as pl
from jax.experimental.pallas import tpu as pltpu
from jax.experimental.pallas import tpu_sc as plsc

_NUM_CORES = 2
_NUM_SUBCORES = 16
_LANES = 16
_NBUF = 3


def _embed_kernel(B, S, E, CW):
    NW = _NUM_CORES * _NUM_SUBCORES
    SP = S // NW
    NH = SP // CW
    NCH = B * NH

    mesh = plsc.VectorSubcoreMesh(core_axis_name="c", subcore_axis_name="s")

    scratch = [
        pltpu.VMEM((B, SP), jnp.int32),
        pltpu.VMEM((SP, E), jnp.float32),
    ]
    scratch += [pltpu.VMEM((CW, E), jnp.float32) for _ in range(_NBUF)]
    scratch += [pltpu.SemaphoreType.DMA for _ in range(2 * _NBUF + 2)]

    def chunk_bh(ch):
        return ch // NH, ch % NH

    @functools.partial(
        pl.kernel,
        mesh=mesh,
        out_type=jax.ShapeDtypeStruct((B, S, E), jnp.float32),
        scratch_types=scratch,
    )
    def k(x_hbm, tok_hbm, pos_hbm, out_hbm, idx_v, pos_v, *bufs):
        rows = bufs[0:_NBUF]
        sg = bufs[_NBUF:2 * _NBUF]
        so = bufs[2 * _NBUF:3 * _NBUF]
        s_idx = bufs[3 * _NBUF]
        s_pos = bufs[3 * _NBUF + 1]

        wid = lax.axis_index("s") * _NUM_CORES + lax.axis_index("c")
        p0 = wid * SP

        for b in range(B):
            pltpu.async_copy(x_hbm.at[b, pl.ds(p0, SP)], idx_v.at[b], s_idx)
        pltpu.async_copy(pos_hbm.at[pl.ds(p0, SP)], pos_v, s_pos)
        for b in range(B):
            pltpu.make_async_copy(
                x_hbm.at[b, pl.ds(p0, SP)], idx_v.at[b], s_idx
            ).wait()

        def start(ch):
            b, h = chunk_bh(ch)
            g = ch % _NBUF
            pltpu.async_copy(
                tok_hbm.at[idx_v.at[b, pl.ds(h * CW, CW)]], rows[g], sg[g]
            )

        def wait_in(ch):
            b, h = chunk_bh(ch)
            g = ch % _NBUF
            pltpu.make_async_copy(
                tok_hbm.at[idx_v.at[b, pl.ds(h * CW, CW)]], rows[g], sg[g]
            ).wait()

        def out_slice(ch):
            b, h = chunk_bh(ch)
            return out_hbm.at[b, pl.ds(p0 + h * CW, CW)]

        def wait_out(ch):
            g = ch % _NBUF
            pltpu.make_async_copy(rows[g], out_slice(ch), so[g]).wait()

        def add_and_store(ch):
            _, h = chunk_bh(ch)
            g = ch % _NBUF

            @pl.loop(0, CW)
            def _pos(p):
                @plsc.parallel_loop(0, E, step=_LANES, unroll=4)
                def _col(e):
                    pv = pos_v.at[h * CW + p, pl.ds(e, _LANES)][...]
                    plsc.addupdate(rows[g].at[p, pl.ds(e, _LANES)], pv)

            pltpu.async_copy(rows[g], out_slice(ch), so[g])

        start(0)
        start(1)
        pltpu.make_async_copy(pos_hbm.at[pl.ds(p0, SP)], pos_v, s_pos).wait()
        for ch in range(NCH):
            wait_in(ch)
            add_and_store(ch)
            if ch + 2 < NCH:
                if ch >= 1:
                    wait_out(ch - 1)
                start(ch + 2)
        wait_out(NCH - 2)
        wait_out(NCH - 1)

    return k


def kernel(x, tok_table, pos_table):
    B, S = x.shape
    _, E = tok_table.shape
    return _embed_kernel(B, S, E, CW=32)(
        x.astype(jnp.int32), tok_table, pos_table
    )


# speedup vs baseline: 2.3611x; 1.0389x over previous
import functools

import jax
import jax.numpy as jnp
from jax import lax
from jax.experimental import pallas as pl
from jax.experimental.pallas import tpu as pltpu
from jax.experimental.pallas import tpu_sc as plsc

_NUM_CORES = 2
_NUM_SUBCORES = 16
_LANES = 16
_NBUF = 6
_LOOKAHEAD = 4


def _embed_kernel(B, S, E, CW):
    NW = _NUM_CORES * _NUM_SUBCORES
    SP = S // NW
    NH = SP // CW
    NCH = B * NH

    mesh = plsc.VectorSubcoreMesh(core_axis_name="c", subcore_axis_name="s")

    scratch = [
        pltpu.VMEM((B, SP), jnp.int32),
        pltpu.VMEM((SP, E), jnp.float32),
    ]
    scratch += [pltpu.VMEM((CW, E), jnp.float32) for _ in range(_NBUF)]
    scratch += [pltpu.SemaphoreType.DMA for _ in range(2 * _NBUF + 2)]

    def chunk_bh(ch):
        return ch // NH, ch % NH

    @functools.partial(
        pl.kernel,
        mesh=mesh,
        out_type=jax.ShapeDtypeStruct((B, S, E), jnp.float32),
        scratch_types=scratch,
    )
    def k(x_hbm, tok_hbm, pos_hbm, out_hbm, idx_v, pos_v, *bufs):
        rows = bufs[0:_NBUF]
        sg = bufs[_NBUF:2 * _NBUF]
        so = bufs[2 * _NBUF:3 * _NBUF]
        s_idx = bufs[3 * _NBUF]
        s_pos = bufs[3 * _NBUF + 1]

        wid = lax.axis_index("s") * _NUM_CORES + lax.axis_index("c")
        p0 = wid * SP

        for b in range(B):
            pltpu.async_copy(x_hbm.at[b, pl.ds(p0, SP)], idx_v.at[b], s_idx)
        pltpu.async_copy(pos_hbm.at[pl.ds(p0, SP)], pos_v, s_pos)
        for b in range(B):
            pltpu.make_async_copy(
                x_hbm.at[b, pl.ds(p0, SP)], idx_v.at[b], s_idx
            ).wait()

        def start(ch):
            b, h = chunk_bh(ch)
            g = ch % _NBUF
            pltpu.async_copy(
                tok_hbm.at[idx_v.at[b, pl.ds(h * CW, CW)]], rows[g], sg[g]
            )

        def wait_in(ch):
            b, h = chunk_bh(ch)
            g = ch % _NBUF
            pltpu.make_async_copy(
                tok_hbm.at[idx_v.at[b, pl.ds(h * CW, CW)]], rows[g], sg[g]
            ).wait()

        def out_slice(ch):
            b, h = chunk_bh(ch)
            return out_hbm.at[b, pl.ds(p0 + h * CW, CW)]

        def wait_out(ch):
            g = ch % _NBUF
            pltpu.make_async_copy(rows[g], out_slice(ch), so[g]).wait()

        def add_and_store(ch):
            _, h = chunk_bh(ch)
            g = ch % _NBUF

            @pl.loop(0, CW)
            def _pos(p):
                @plsc.parallel_loop(0, E, step=_LANES, unroll=4)
                def _col(e):
                    pv = pos_v.at[h * CW + p, pl.ds(e, _LANES)][...]
                    plsc.addupdate(rows[g].at[p, pl.ds(e, _LANES)], pv)

            pltpu.async_copy(rows[g], out_slice(ch), so[g])

        for ch in range(min(_LOOKAHEAD, NCH)):
            start(ch)
        pltpu.make_async_copy(pos_hbm.at[pl.ds(p0, SP)], pos_v, s_pos).wait()
        for ch in range(NCH):
            wait_in(ch)
            add_and_store(ch)
            n = ch + _LOOKAHEAD
            if n < NCH:
                if n - _NBUF >= 0:
                    wait_out(n - _NBUF)
                start(n)
        for ch in range(max(0, NCH - _NBUF), NCH):
            wait_out(ch)

    return k


def kernel(x, tok_table, pos_table):
    B, S = x.shape
    _, E = tok_table.shape
    return _embed_kernel(B, S, E, CW=16)(
        x.astype(jnp.int32), tok_table, pos_table
    )
